# speedup vs baseline: 3.9923x; 3.9923x over previous
import functools
import math

import jax
import jax.numpy as jnp
from jax import lax
from jax.experimental import pallas as pl
from jax.experimental.pallas import tpu as pltpu
from jax.experimental.pallas import tpu_sc as plsc

_B = 16384
_NEWS_D = 768
_CAT_D = 128
_OUT_D = 256

_NC = 2
_NS = 16
_NW = _NC * _NS
_BPW = _B // _NW
_NCH = 4
_CH = _BPW // _NCH


@functools.partial(
    pl.kernel,
    out_type=[
        jax.ShapeDtypeStruct((_B, _NEWS_D), jnp.float32),
        jax.ShapeDtypeStruct((_B, _CAT_D), jnp.float32),
        jax.ShapeDtypeStruct((_B, _CAT_D), jnp.float32),
    ],
    mesh=plsc.VectorSubcoreMesh(core_axis_name="c", subcore_axis_name="s"),
    scratch_types=[
        pltpu.VMEM((_BPW,), jnp.int32),
        pltpu.VMEM((_BPW,), jnp.int32),
        pltpu.VMEM((_BPW,), jnp.int32),
        pltpu.VMEM((_CH, _NEWS_D), jnp.float32),
        pltpu.VMEM((_CH, _CAT_D), jnp.float32),
        pltpu.SemaphoreType.DMA,
    ],
)
def _sc_gather(news_table_h, cat_table_h, sub_table_h, nid_h, cid_h, sid_h,
               news_out, cat_out, sub_out,
               nid_v, cid_v, sid_v, nrows, crows, sem):
    wid = lax.axis_index("s") * _NC + lax.axis_index("c")
    base = wid * _BPW
    pltpu.sync_copy(nid_h.at[pl.ds(base, _BPW)], nid_v)
    pltpu.sync_copy(cid_h.at[pl.ds(base, _BPW)], cid_v)
    pltpu.sync_copy(sid_h.at[pl.ds(base, _BPW)], sid_v)
    for j in range(_NCH):
        pltpu.async_copy(
            news_table_h.at[nid_v.at[pl.ds(j * _CH, _CH)]], nrows, sem).wait()
        pltpu.sync_copy(nrows, news_out.at[pl.ds(base + j * _CH, _CH)])
    for j in range(_NCH):
        pltpu.async_copy(
            cat_table_h.at[cid_v.at[pl.ds(j * _CH, _CH)]], crows, sem).wait()
        pltpu.sync_copy(crows, cat_out.at[pl.ds(base + j * _CH, _CH)])
    for j in range(_NCH):
        pltpu.async_copy(
            sub_table_h.at[sid_v.at[pl.ds(j * _CH, _CH)]], crows, sem).wait()
        pltpu.sync_copy(crows, sub_out.at[pl.ds(base + j * _CH, _CH)])


def _gelu_tanh(x):
    c0 = math.sqrt(2.0 / math.pi)
    return 0.5 * x * (1.0 + jnp.tanh(c0 * (x + 0.044715 * x * x * x)))


def _tc_body(n_ref, c_ref, s_ref, w1_ref, w2_ref, w3_ref, b_ref, o_ref):
    acc = jnp.dot(n_ref[...], w1_ref[...], preferred_element_type=jnp.float32)
    acc = acc + jnp.dot(c_ref[...], w2_ref[...], preferred_element_type=jnp.float32)
    acc = acc + jnp.dot(s_ref[...], w3_ref[...], preferred_element_type=jnp.float32)
    acc = acc + b_ref[...]
    o_ref[...] = _gelu_tanh(acc)


_BM = 512


def _tc_fused(news_g, cat_g, sub_g, w1, w2, w3, b2):
    return pl.pallas_call(
        _tc_body,
        grid=(_B // _BM,),
        in_specs=[
            pl.BlockSpec((_BM, _NEWS_D), lambda i: (i, 0)),
            pl.BlockSpec((_BM, _CAT_D), lambda i: (i, 0)),
            pl.BlockSpec((_BM, _CAT_D), lambda i: (i, 0)),
            pl.BlockSpec((_NEWS_D, _OUT_D), lambda i: (0, 0)),
            pl.BlockSpec((_CAT_D, _OUT_D), lambda i: (0, 0)),
            pl.BlockSpec((_CAT_D, _OUT_D), lambda i: (0, 0)),
            pl.BlockSpec((1, _OUT_D), lambda i: (0, 0)),
        ],
        out_specs=pl.BlockSpec((_BM, _OUT_D), lambda i: (i, 0)),
        out_shape=jax.ShapeDtypeStruct((_B, _OUT_D), jnp.float32),
        compiler_params=pltpu.CompilerParams(
            dimension_semantics=("arbitrary",)),
    )(news_g, cat_g, sub_g, w1, w2, w3, b2)


def kernel(news_ids, news_categ, news_subcateg, news_table, cat_table,
           subcat_table, W, b):
    nid = news_ids.astype(jnp.int32)
    cid = news_categ.astype(jnp.int32)
    sid = news_subcateg.astype(jnp.int32)
    news_g, cat_g, sub_g = _sc_gather(news_table, cat_table, subcat_table,
                                      nid, cid, sid)
    w1 = W[:_NEWS_D]
    w2 = W[_NEWS_D:_NEWS_D + _CAT_D]
    w3 = W[_NEWS_D + _CAT_D:]
    b2 = b.reshape(1, _OUT_D)
    return _tc_fused(news_g, cat_g, sub_g, w1, w2, w3, b2)


# speedup vs baseline: 4.0840x; 1.0230x over previous
import functools
import math

import jax
import jax.numpy as jnp
from jax import lax
from jax.experimental import pallas as pl
from jax.experimental.pallas import tpu as pltpu
from jax.experimental.pallas import tpu_sc as plsc

_B = 16384
_NEWS_D = 768
_CAT_D = 128
_OUT_D = 256

_NC = 2
_NS = 16
_NW = _NC * _NS
_BPW = _B // _NW
_NCH = 8
_CH = _BPW // _NCH


@functools.partial(
    pl.kernel,
    out_type=[
        jax.ShapeDtypeStruct((_B, _NEWS_D), jnp.float32),
        jax.ShapeDtypeStruct((_B, _CAT_D), jnp.float32),
        jax.ShapeDtypeStruct((_B, _CAT_D), jnp.float32),
    ],
    mesh=plsc.VectorSubcoreMesh(core_axis_name="c", subcore_axis_name="s"),
    scratch_types=[
        pltpu.VMEM((_BPW,), jnp.int32),
        pltpu.VMEM((_BPW,), jnp.int32),
        pltpu.VMEM((_BPW,), jnp.int32),
        pltpu.VMEM((_CH, _NEWS_D), jnp.float32),
        pltpu.VMEM((_CH, _NEWS_D), jnp.float32),
        pltpu.VMEM((_CH, _CAT_D), jnp.float32),
        pltpu.VMEM((_CH, _CAT_D), jnp.float32),
        pltpu.SemaphoreType.DMA,
        pltpu.SemaphoreType.DMA,
    ],
)
def _sc_gather(news_table_h, cat_table_h, sub_table_h, nid_h, cid_h, sid_h,
               news_out, cat_out, sub_out,
               nid_v, cid_v, sid_v, nb0, nb1, cb0, cb1, sem0, sem1):
    wid = lax.axis_index("s") * _NC + lax.axis_index("c")
    base = wid * _BPW
    pltpu.sync_copy(nid_h.at[pl.ds(base, _BPW)], nid_v)
    pltpu.sync_copy(cid_h.at[pl.ds(base, _BPW)], cid_v)
    pltpu.sync_copy(sid_h.at[pl.ds(base, _BPW)], sid_v)

    def run(table_h, idx_v, out_h, bufs, sems):
        def fire(j):
            pltpu.async_copy(
                table_h.at[idx_v.at[pl.ds(j * _CH, _CH)]],
                bufs[j % 2], sems[j % 2])
        fire(0)
        fire(1)
        for j in range(_NCH):
            pltpu.make_async_copy(
                table_h.at[idx_v.at[pl.ds(j * _CH, _CH)]],
                bufs[j % 2], sems[j % 2]).wait()
            pltpu.sync_copy(bufs[j % 2], out_h.at[pl.ds(base + j * _CH, _CH)])
            if j + 2 < _NCH:
                fire(j + 2)

    run(news_table_h, nid_v, news_out, (nb0, nb1), (sem0, sem1))
    run(cat_table_h, cid_v, cat_out, (cb0, cb1), (sem0, sem1))
    run(sub_table_h, sid_v, sub_out, (cb0, cb1), (sem0, sem1))


def _gelu_tanh(x):
    c0 = math.sqrt(2.0 / math.pi)
    return 0.5 * x * (1.0 + jnp.tanh(c0 * (x + 0.044715 * x * x * x)))


def _tc_body(n_ref, c_ref, s_ref, w1_ref, w2_ref, w3_ref, b_ref, o_ref):
    acc = jnp.dot(n_ref[...], w1_ref[...], preferred_element_type=jnp.float32)
    acc = acc + jnp.dot(c_ref[...], w2_ref[...], preferred_element_type=jnp.float32)
    acc = acc + jnp.dot(s_ref[...], w3_ref[...], preferred_element_type=jnp.float32)
    acc = acc + b_ref[...]
    o_ref[...] = _gelu_tanh(acc)


_BM = 512


def _tc_fused(news_g, cat_g, sub_g, w1, w2, w3, b2):
    return pl.pallas_call(
        _tc_body,
        grid=(_B // _BM,),
        in_specs=[
            pl.BlockSpec((_BM, _NEWS_D), lambda i: (i, 0)),
            pl.BlockSpec((_BM, _CAT_D), lambda i: (i, 0)),
            pl.BlockSpec((_BM, _CAT_D), lambda i: (i, 0)),
            pl.BlockSpec((_NEWS_D, _OUT_D), lambda i: (0, 0)),
            pl.BlockSpec((_CAT_D, _OUT_D), lambda i: (0, 0)),
            pl.BlockSpec((_CAT_D, _OUT_D), lambda i: (0, 0)),
            pl.BlockSpec((1, _OUT_D), lambda i: (0, 0)),
        ],
        out_specs=pl.BlockSpec((_BM, _OUT_D), lambda i: (i, 0)),
        out_shape=jax.ShapeDtypeStruct((_B, _OUT_D), jnp.float32),
        compiler_params=pltpu.CompilerParams(
            dimension_semantics=("arbitrary",)),
    )(news_g, cat_g, sub_g, w1, w2, w3, b2)


def kernel(news_ids, news_categ, news_subcateg, news_table, cat_table,
           subcat_table, W, b):
    nid = news_ids.astype(jnp.int32)
    cid = news_categ.astype(jnp.int32)
    sid = news_subcateg.astype(jnp.int32)
    news_g, cat_g, sub_g = _sc_gather(news_table, cat_table, subcat_table,
                                      nid, cid, sid)
    w1 = W[:_NEWS_D]
    w2 = W[_NEWS_D:_NEWS_D + _CAT_D]
    w3 = W[_NEWS_D + _CAT_D:]
    b2 = b.reshape(1, _OUT_D)
    return _tc_fused(news_g, cat_g, sub_g, w1, w2, w3, b2)
